# baseline (speedup 1.0000x reference)
import functools

import jax
import jax.numpy as jnp
from jax import lax
from jax.experimental import pallas as pl
from jax.experimental.pallas import tpu as pltpu
from jax.experimental.pallas import tpu_sc as plsc

_B = 16384
_D = 32
_FEAT = 102
_OUT = 64
_ISIZE = 83
_CHUNK = 128
_BBLK = 2048


def _make_sc_gather():
    info = plsc.get_sparse_core_info()
    nc, ns = info.num_cores, info.num_subcores
    nw = nc * ns
    rows_per_w = _B // nw
    n_chunks = rows_per_w // _CHUNK
    mesh = plsc.VectorSubcoreMesh(core_axis_name="c", subcore_axis_name="s")

    @functools.partial(
        pl.kernel,
        mesh=mesh,
        out_type=jax.ShapeDtypeStruct((_B, _D), jnp.float32),
        scratch_types=[
            pltpu.VMEM((n_chunks, _CHUNK), jnp.int32),
            pltpu.VMEM((rows_per_w, _D), jnp.float32),
            pltpu.SemaphoreType.DMA,
        ],
        compiler_params=pltpu.CompilerParams(use_tc_tiling_on_sc=False),
    )
    def sc_gather(table_hbm, idx_hbm, out_hbm, idx_v, rows_v, sem):
        wid = lax.axis_index("s") * nc + lax.axis_index("c")
        pltpu.sync_copy(idx_hbm.at[pl.ds(wid * n_chunks, n_chunks)], idx_v)
        copies = [
            pltpu.async_copy(
                table_hbm.at[idx_v.at[j]],
                rows_v.at[pl.ds(j * _CHUNK, _CHUNK)],
                sem,
            )
            for j in range(n_chunks)
        ]
        for c in copies:
            c.wait()
        pltpu.sync_copy(rows_v, out_hbm.at[pl.ds(wid * rows_per_w, rows_per_w)])

    return sc_gather, nw * n_chunks


_SC_GATHER, _IDX_ROWS = _make_sc_gather()


def _tc_body(x_ref, ne_ref, it_ref, w_ref, b_ref, o_ref):
    xb = x_ref[:, 10:93]
    m = jnp.max(xb, axis=1, keepdims=True)
    colid = lax.broadcasted_iota(jnp.int32, xb.shape, 1)
    iidx = jnp.min(jnp.where(xb == m, colid, _ISIZE), axis=1)
    onehot = (colid == iidx[:, None]).astype(jnp.float32)
    ie = jnp.dot(onehot, it_ref[...], preferred_element_type=jnp.float32)
    x64 = jnp.concatenate([ne_ref[...], ie], axis=1)
    y = lax.dot_general(
        x64, w_ref[...], (((1,), (1,)), ((), ())),
        preferred_element_type=jnp.float32,
    )
    o_ref[...] = y + b_ref[...] + x64


@jax.jit
def kernel(X, numeric_table, i_table, W_fc, b_fc):
    idx = X[:, 9].reshape(_IDX_ROWS, _CHUNK)
    ne = _SC_GATHER(numeric_table, idx)
    grid = _B // _BBLK
    return pl.pallas_call(
        _tc_body,
        grid=(grid,),
        in_specs=[
            pl.BlockSpec((_BBLK, _FEAT), lambda i: (i, 0)),
            pl.BlockSpec((_BBLK, _D), lambda i: (i, 0)),
            pl.BlockSpec((_ISIZE, _D), lambda i: (0, 0)),
            pl.BlockSpec((_OUT, _OUT), lambda i: (0, 0)),
            pl.BlockSpec((1, _OUT), lambda i: (0, 0)),
        ],
        out_specs=pl.BlockSpec((_BBLK, _OUT), lambda i: (i, 0)),
        out_shape=jax.ShapeDtypeStruct((_B, _OUT), jnp.float32),
    )(X, ne, i_table, W_fc, b_fc.reshape(1, _OUT))


# speedup vs baseline: 1.0412x; 1.0412x over previous
import functools

import jax
import jax.numpy as jnp
from jax import lax
from jax.experimental import pallas as pl
from jax.experimental.pallas import tpu as pltpu
from jax.experimental.pallas import tpu_sc as plsc

_B = 16384
_D = 32
_FEAT = 102
_OUT = 64
_BBLK = 2048
_CHUNK = 128


def _make_sc_gather():
    info = plsc.get_sparse_core_info()
    nc, ns = info.num_cores, info.num_subcores
    nw = nc * ns
    rpw = _B // nw
    n_chunks = rpw // _CHUNK
    mesh = plsc.VectorSubcoreMesh(core_axis_name="c", subcore_axis_name="s")

    @functools.partial(
        pl.kernel,
        mesh=mesh,
        out_type=jax.ShapeDtypeStruct((_B, _D), jnp.float32),
        scratch_types=[
            pltpu.VMEM((n_chunks, _CHUNK), jnp.int32),
            pltpu.VMEM((rpw, _D), jnp.float32),
            pltpu.SemaphoreType.DMA,
        ],
        compiler_params=pltpu.CompilerParams(use_tc_tiling_on_sc=False),
    )
    def sc_gather(table_hbm, idx_hbm, out_hbm, idx_v, rows_v, sem):
        wid = lax.axis_index("s") * nc + lax.axis_index("c")
        pltpu.sync_copy(idx_hbm.at[pl.ds(wid * n_chunks, n_chunks)], idx_v)
        copies = [
            pltpu.async_copy(
                table_hbm.at[idx_v.at[j]],
                rows_v.at[pl.ds(j * _CHUNK, _CHUNK)],
                sem,
            )
            for j in range(n_chunks)
        ]
        for c in copies:
            c.wait()
        pltpu.sync_copy(rows_v, out_hbm.at[pl.ds(wid * rpw, rpw)])

    return sc_gather, nw * n_chunks


_SC_GATHER, _IDX_ROWS = _make_sc_gather()


def _tc_a_body(x_ref, c_ref, b_ref, o_ref):
    xb = x_ref[...]
    ri = lax.broadcasted_iota(jnp.int32, xb.shape, 0)
    valid = (ri >= 10) & (ri <= 92)
    xm = jnp.where(valid, xb, jnp.int32(-2147483648))
    m = jnp.max(xm, axis=0, keepdims=True)
    iidx = jnp.min(jnp.where(xm == m, ri, _FEAT), axis=0, keepdims=True)
    onehot = (ri == iidx).astype(jnp.float32)
    o_ref[...] = (
        jnp.dot(c_ref[...], onehot, preferred_element_type=jnp.float32)
        + b_ref[...]
    )


def _tc_b_body(p_ref, ne_ref, w_ref, o_ref):
    o_ref[...] = (
        lax.dot_general(
            w_ref[...],
            ne_ref[...],
            (((1,), (1,)), ((), ())),
            preferred_element_type=jnp.float32,
        )
        + p_ref[...]
    )


@jax.jit
def kernel(X, numeric_table, i_table, W_fc, b_fc):
    Xt = X.T
    itT_pad = jnp.pad(i_table.T, ((0, 0), (10, _FEAT - 93)))
    Wi = W_fc[:, _D:]
    Wn = W_fc[:, :_D]
    C = Wi @ itT_pad + jnp.concatenate(
        [jnp.zeros((_D, _FEAT), jnp.float32), itT_pad], axis=0
    )
    Wn_aug = Wn + jnp.concatenate(
        [jnp.eye(_D, dtype=jnp.float32), jnp.zeros((_D, _D), jnp.float32)],
        axis=0,
    )

    idx = X[:, 9].reshape(_IDX_ROWS, _CHUNK)
    ne = _SC_GATHER(numeric_table, idx)

    grid = _B // _BBLK
    partialT = pl.pallas_call(
        _tc_a_body,
        grid=(grid,),
        in_specs=[
            pl.BlockSpec((_FEAT, _BBLK), lambda i: (0, i)),
            pl.BlockSpec((_OUT, _FEAT), lambda i: (0, 0)),
            pl.BlockSpec((_OUT, 1), lambda i: (0, 0)),
        ],
        out_specs=pl.BlockSpec((_OUT, _BBLK), lambda i: (0, i)),
        out_shape=jax.ShapeDtypeStruct((_OUT, _B), jnp.float32),
    )(Xt, C, b_fc.reshape(_OUT, 1))

    outT = pl.pallas_call(
        _tc_b_body,
        grid=(grid,),
        in_specs=[
            pl.BlockSpec((_OUT, _BBLK), lambda i: (0, i)),
            pl.BlockSpec((_BBLK, _D), lambda i: (i, 0)),
            pl.BlockSpec((_OUT, _D), lambda i: (0, 0)),
        ],
        out_specs=pl.BlockSpec((_OUT, _BBLK), lambda i: (0, i)),
        out_shape=jax.ShapeDtypeStruct((_OUT, _B), jnp.float32),
    )(partialT, ne, Wn_aug)
    return outT.T
